# baseline (speedup 1.0000x reference)
import functools

import jax
import jax.numpy as jnp
from jax import lax
from jax.experimental import pallas as pl
from jax.experimental.pallas import tpu as pltpu
from jax.experimental.pallas import tpu_sc as plsc

NC = 2
NS = 16
L = 16
NW = NC * NS

BATCH = 16384
D = 64
N_ROWS = 1000000
PAIR = 2 * D
NPAIR = ((N_ROWS + 255) // 256) * 128
B_PER_W = BATCH // NW
CHUNK = 128
NCHUNK = B_PER_W // CHUNK
GROUP = 16
NBUF = 2

WB2 = 4096
WIN = 2 * WB2
NB = (NPAIR + WB2 - 1) // WB2


def _pack_body(t_ref, o_ref):
    SUB = WIN // 4
    for s in range(4):
        tT = jnp.transpose(t_ref[:, pl.ds(s * SUB, SUB)])
        halves = tT.reshape(SUB // 256, 2, 128, D)
        r0 = s * (SUB // 2)
        o_ref[pl.ds(r0, SUB // 2), 0:D] = halves[:, 0].reshape(SUB // 2, D)
        o_ref[pl.ds(r0, SUB // 2), D:PAIR] = halves[:, 1].reshape(SUB // 2, D)


def _pack(table_t):
    return pl.pallas_call(
        _pack_body,
        grid=(NB,),
        in_specs=[pl.BlockSpec((D, WIN), lambda i: (0, i))],
        out_specs=pl.BlockSpec((WB2, PAIR), lambda i: (i, 0)),
        out_shape=jax.ShapeDtypeStruct((NPAIR, PAIR), jnp.float32),
        compiler_params=pltpu.CompilerParams(
            dimension_semantics=("parallel",)),
    )(table_t)


def _sc_body(users_hbm, items_hbm, uemb_hbm, iemb_hbm, out_hbm,
             uidx_v, iidx_v, upr_v, ipr_v, u_v, i_v, out_v, sem0, sem1):
    wid = lax.axis_index("s") * NC + lax.axis_index("c")
    sems = (sem0, sem1)

    pltpu.sync_copy(users_hbm.at[pl.ds(wid * NCHUNK, NCHUNK)], uidx_v)
    pltpu.sync_copy(items_hbm.at[pl.ds(wid * NCHUNK, NCHUNK)], iidx_v)

    def _staging_row_u(v):
        return jnp.bitwise_or(
            lax.shift_left(lax.shift_right_logical(v, 8), 7),
            jnp.bitwise_and(v, 127))

    for k in range(NCHUNK):
        for c in range(CHUNK // L):
            sl = pl.ds(c * L, L)
            upr_v[k, sl] = _staging_row_u(uidx_v[k, sl])
            ipr_v[k, sl] = lax.shift_right_logical(iidx_v[k, sl], 1)

    def fire(k):
        b = k % NBUF
        return (pltpu.async_copy(uemb_hbm.at[upr_v.at[k]], u_v.at[b], sems[b]),
                pltpu.async_copy(iemb_hbm.at[ipr_v.at[k]], i_v.at[b], sems[b]))

    inflight = {0: fire(0)}
    lanes = lax.iota(jnp.int32, L)

    for k in range(NCHUNK):
        for cp in inflight.pop(k):
            cp.wait()
        if k + 1 < NCHUNK:
            inflight[k + 1] = fire(k + 1)
        b = k % NBUF
        bvec = jnp.full((L,), b, jnp.int32)

        @pl.loop(0, CHUNK, step=GROUP)
        def _(r0, k=k, bvec=bvec):
            rvec = r0 + lanes
            uoff = lax.shift_left(jnp.bitwise_and(
                lax.shift_right_logical(uidx_v[k, pl.ds(r0, L)], 7), 1), 6)
            ioff = lax.shift_left(jnp.bitwise_and(
                iidx_v[k, pl.ds(r0, L)], 1), 6)
            acc = jnp.zeros((L,), jnp.float32)
            for m in range(D):
                ucol = plsc.load_gather(u_v, [bvec, rvec, uoff + m])
                icol = plsc.load_gather(i_v, [bvec, rvec, ioff + m])
                acc = acc + ucol * icol
            probs = 1.0 / (1.0 + jnp.exp(-acc))
            out_v[pl.ds(k * CHUNK + r0, GROUP)] = probs

    pltpu.sync_copy(out_v, out_hbm.at[pl.ds(wid * B_PER_W, B_PER_W)])


_cp = pltpu.CompilerParams(needs_layout_passes=False)


@functools.partial(
    pl.kernel,
    compiler_params=_cp,
    out_type=jax.ShapeDtypeStruct((BATCH,), jnp.float32),
    mesh=plsc.VectorSubcoreMesh(core_axis_name="c", subcore_axis_name="s"),
    scratch_types=[
        pltpu.VMEM((NCHUNK, CHUNK), jnp.int32),
        pltpu.VMEM((NCHUNK, CHUNK), jnp.int32),
        pltpu.VMEM((NCHUNK, CHUNK), jnp.int32),
        pltpu.VMEM((NCHUNK, CHUNK), jnp.int32),
        pltpu.VMEM((NBUF, CHUNK, PAIR), jnp.float32),
        pltpu.VMEM((NBUF, CHUNK, PAIR), jnp.float32),
        pltpu.VMEM((B_PER_W,), jnp.float32),
        pltpu.SemaphoreType.DMA,
        pltpu.SemaphoreType.DMA,
    ],
)
def _sc_call(users_hbm, items_hbm, uemb_hbm, iemb_hbm, out_hbm,
             uidx_v, iidx_v, upr_v, ipr_v, u_v, i_v, out_v, sem0, sem1):
    _sc_body(users_hbm, items_hbm, uemb_hbm, iemb_hbm, out_hbm,
             uidx_v, iidx_v, upr_v, ipr_v, u_v, i_v, out_v, sem0, sem1)


def kernel(users, items, user_emb, item_emb):
    users2 = users.reshape(NW * NCHUNK, CHUNK)
    items2 = items.reshape(NW * NCHUNK, CHUNK)
    s_u = _pack(user_emb.T)
    s_i = item_emb.reshape(N_ROWS // 2, PAIR)
    return _sc_call(users2, items2, s_u, s_i)


# speedup vs baseline: 1.3782x; 1.3782x over previous
import functools

import jax
import jax.numpy as jnp
from jax import lax
from jax.experimental import pallas as pl
from jax.experimental.pallas import tpu as pltpu
from jax.experimental.pallas import tpu_sc as plsc

NC = 2
NS = 16
L = 16
NW = NC * NS

BATCH = 16384
D = 64
N_ROWS = 1000000
PAIR = 2 * D
NB = 123
NPAIR = NB * 4096
B_PER_W = BATCH // NW
CHUNK = 128
NCHUNK = B_PER_W // CHUNK
GROUP = 16
NBUF = 2

WB2 = 4096
WIN = 2 * WB2


def _pack_body(t_ref, o_ref):
    tT = jnp.transpose(t_ref[...])
    o_ref[:, 0:D] = tT[0:WB2]
    o_ref[:, D:PAIR] = tT[WB2:WIN]


def _pack(table_t):
    return pl.pallas_call(
        _pack_body,
        grid=(NB,),
        in_specs=[pl.BlockSpec((D, WIN), lambda i: (0, i))],
        out_specs=pl.BlockSpec((WB2, PAIR), lambda i: (i, 0)),
        out_shape=jax.ShapeDtypeStruct((NPAIR, PAIR), jnp.float32),
        compiler_params=pltpu.CompilerParams(
            dimension_semantics=("parallel",)),
    )(table_t)


def _sc_body(users_hbm, items_hbm, uemb_hbm, iemb_hbm, out_hbm,
             uidx_v, iidx_v, upr_v, ipr_v, u_v, i_v, out_v, sem0, sem1):
    wid = lax.axis_index("s") * NC + lax.axis_index("c")
    sems = (sem0, sem1)

    pltpu.sync_copy(users_hbm.at[pl.ds(wid * NCHUNK, NCHUNK)], uidx_v)
    pltpu.sync_copy(items_hbm.at[pl.ds(wid * NCHUNK, NCHUNK)], iidx_v)

    def _staging_row_u(v):
        return jnp.bitwise_or(
            lax.shift_left(lax.shift_right_logical(v, 13), 12),
            jnp.bitwise_and(v, 4095))

    for k in range(NCHUNK):
        for c in range(CHUNK // L):
            sl = pl.ds(c * L, L)
            upr_v[k, sl] = _staging_row_u(uidx_v[k, sl])
            ipr_v[k, sl] = _staging_row_u(iidx_v[k, sl])

    def fire(k):
        b = k % NBUF
        return (pltpu.async_copy(uemb_hbm.at[upr_v.at[k]], u_v.at[b], sems[b]),
                pltpu.async_copy(iemb_hbm.at[ipr_v.at[k]], i_v.at[b], sems[b]))

    inflight = {0: fire(0)}
    lanes = lax.iota(jnp.int32, L)

    for k in range(NCHUNK):
        for cp in inflight.pop(k):
            cp.wait()
        if k + 1 < NCHUNK:
            inflight[k + 1] = fire(k + 1)
        b = k % NBUF
        bvec = jnp.full((L,), b, jnp.int32)

        @pl.loop(0, CHUNK, step=GROUP)
        def _(r0, k=k, bvec=bvec):
            rvec = r0 + lanes
            uoff = lax.shift_left(jnp.bitwise_and(
                lax.shift_right_logical(uidx_v[k, pl.ds(r0, L)], 12), 1), 6)
            ioff = lax.shift_left(jnp.bitwise_and(
                lax.shift_right_logical(iidx_v[k, pl.ds(r0, L)], 12), 1), 6)
            acc = jnp.zeros((L,), jnp.float32)
            for m in range(D):
                ucol = plsc.load_gather(u_v, [bvec, rvec, uoff + m])
                icol = plsc.load_gather(i_v, [bvec, rvec, ioff + m])
                acc = acc + ucol * icol
            probs = 1.0 / (1.0 + jnp.exp(-acc))
            out_v[pl.ds(k * CHUNK + r0, GROUP)] = probs

    pltpu.sync_copy(out_v, out_hbm.at[pl.ds(wid * B_PER_W, B_PER_W)])


_cp = pltpu.CompilerParams(needs_layout_passes=False)


@functools.partial(
    pl.kernel,
    compiler_params=_cp,
    out_type=jax.ShapeDtypeStruct((BATCH,), jnp.float32),
    mesh=plsc.VectorSubcoreMesh(core_axis_name="c", subcore_axis_name="s"),
    scratch_types=[
        pltpu.VMEM((NCHUNK, CHUNK), jnp.int32),
        pltpu.VMEM((NCHUNK, CHUNK), jnp.int32),
        pltpu.VMEM((NCHUNK, CHUNK), jnp.int32),
        pltpu.VMEM((NCHUNK, CHUNK), jnp.int32),
        pltpu.VMEM((NBUF, CHUNK, PAIR), jnp.float32),
        pltpu.VMEM((NBUF, CHUNK, PAIR), jnp.float32),
        pltpu.VMEM((B_PER_W,), jnp.float32),
        pltpu.SemaphoreType.DMA,
        pltpu.SemaphoreType.DMA,
    ],
)
def _sc_call(users_hbm, items_hbm, uemb_hbm, iemb_hbm, out_hbm,
             uidx_v, iidx_v, upr_v, ipr_v, u_v, i_v, out_v, sem0, sem1):
    _sc_body(users_hbm, items_hbm, uemb_hbm, iemb_hbm, out_hbm,
             uidx_v, iidx_v, upr_v, ipr_v, u_v, i_v, out_v, sem0, sem1)


def kernel(users, items, user_emb, item_emb):
    users2 = users.reshape(NW * NCHUNK, CHUNK)
    items2 = items.reshape(NW * NCHUNK, CHUNK)
    s_u = _pack(user_emb.T)
    s_i = _pack(item_emb.T)
    return _sc_call(users2, items2, s_u, s_i)


# speedup vs baseline: 1.3793x; 1.0008x over previous
import functools

import jax
import jax.numpy as jnp
from jax import lax
from jax.experimental import pallas as pl
from jax.experimental.pallas import tpu as pltpu
from jax.experimental.pallas import tpu_sc as plsc

NC = 2
NS = 16
L = 16
NW = NC * NS

BATCH = 16384
D = 64
N_ROWS = 1000000
PAIR = 2 * D
NB = 123
NPAIR = NB * 4096
B_PER_W = BATCH // NW
CHUNK = 128
NCHUNK = B_PER_W // CHUNK
GROUP = 16
NBUF = 2

WB2 = 4096
WIN = 2 * WB2


def _pack_body(t_ref, o_ref):
    tT = jnp.transpose(t_ref[...])
    o_ref[:, 0:D] = tT[0:WB2]
    o_ref[:, D:PAIR] = tT[WB2:WIN]


def _pack(table_t):
    return pl.pallas_call(
        _pack_body,
        grid=(NB,),
        in_specs=[pl.BlockSpec((D, WIN), lambda i: (0, i))],
        out_specs=pl.BlockSpec((WB2, PAIR), lambda i: (i, 0)),
        out_shape=jax.ShapeDtypeStruct((NPAIR, PAIR), jnp.float32),
        compiler_params=pltpu.CompilerParams(
            dimension_semantics=("parallel",)),
    )(table_t)


def _sc_body(users_hbm, items_hbm, uemb_hbm, iemb_hbm, out_hbm,
             uidx_v, iidx_v, upr_v, ipr_v, u_v, i_v, out_v, sem0, sem1):
    wid = lax.axis_index("s") * NC + lax.axis_index("c")
    sems = (sem0, sem1)

    pltpu.sync_copy(users_hbm.at[pl.ds(wid * NCHUNK, NCHUNK)], uidx_v)
    pltpu.sync_copy(items_hbm.at[pl.ds(wid * NCHUNK, NCHUNK)], iidx_v)

    def _staging_row_u(v):
        return jnp.bitwise_or(
            lax.shift_left(lax.shift_right_logical(v, 13), 12),
            jnp.bitwise_and(v, 4095))

    for k in range(NCHUNK):
        for c in range(CHUNK // L):
            sl = pl.ds(c * L, L)
            upr_v[k, sl] = _staging_row_u(uidx_v[k, sl])
            ipr_v[k, sl] = _staging_row_u(iidx_v[k, sl])

    def fire(k):
        b = k % NBUF
        return (pltpu.async_copy(uemb_hbm.at[upr_v.at[k]], u_v.at[b], sems[b]),
                pltpu.async_copy(iemb_hbm.at[ipr_v.at[k]], i_v.at[b], sems[b]))

    inflight = {0: fire(0)}
    lanes = lax.iota(jnp.int32, L)

    for k in range(NCHUNK):
        for cp in inflight.pop(k):
            cp.wait()
        if k + 1 < NCHUNK:
            inflight[k + 1] = fire(k + 1)
        b = k % NBUF
        bvec = jnp.full((L,), b, jnp.int32)

        @pl.loop(0, CHUNK, step=GROUP)
        def _(r0, k=k, bvec=bvec):
            rvec = r0 + lanes
            uoff = lax.shift_left(jnp.bitwise_and(
                lax.shift_right_logical(uidx_v[k, pl.ds(r0, L)], 12), 1), 6)
            ioff = lax.shift_left(jnp.bitwise_and(
                lax.shift_right_logical(iidx_v[k, pl.ds(r0, L)], 12), 1), 6)
            accs = [jnp.zeros((L,), jnp.float32) for _ in range(4)]
            for m in range(D):
                ucol = plsc.load_gather(u_v, [bvec, rvec, uoff + m])
                icol = plsc.load_gather(i_v, [bvec, rvec, ioff + m])
                accs[m % 4] = accs[m % 4] + ucol * icol
            acc = (accs[0] + accs[1]) + (accs[2] + accs[3])
            probs = 1.0 / (1.0 + jnp.exp(-acc))
            out_v[pl.ds(k * CHUNK + r0, GROUP)] = probs

    pltpu.sync_copy(out_v, out_hbm.at[pl.ds(wid * B_PER_W, B_PER_W)])


_cp = pltpu.CompilerParams(needs_layout_passes=False)


@functools.partial(
    pl.kernel,
    compiler_params=_cp,
    out_type=jax.ShapeDtypeStruct((BATCH,), jnp.float32),
    mesh=plsc.VectorSubcoreMesh(core_axis_name="c", subcore_axis_name="s"),
    scratch_types=[
        pltpu.VMEM((NCHUNK, CHUNK), jnp.int32),
        pltpu.VMEM((NCHUNK, CHUNK), jnp.int32),
        pltpu.VMEM((NCHUNK, CHUNK), jnp.int32),
        pltpu.VMEM((NCHUNK, CHUNK), jnp.int32),
        pltpu.VMEM((NBUF, CHUNK, PAIR), jnp.float32),
        pltpu.VMEM((NBUF, CHUNK, PAIR), jnp.float32),
        pltpu.VMEM((B_PER_W,), jnp.float32),
        pltpu.SemaphoreType.DMA,
        pltpu.SemaphoreType.DMA,
    ],
)
def _sc_call(users_hbm, items_hbm, uemb_hbm, iemb_hbm, out_hbm,
             uidx_v, iidx_v, upr_v, ipr_v, u_v, i_v, out_v, sem0, sem1):
    _sc_body(users_hbm, items_hbm, uemb_hbm, iemb_hbm, out_hbm,
             uidx_v, iidx_v, upr_v, ipr_v, u_v, i_v, out_v, sem0, sem1)


def kernel(users, items, user_emb, item_emb):
    users2 = users.reshape(NW * NCHUNK, CHUNK)
    items2 = items.reshape(NW * NCHUNK, CHUNK)
    s_u = _pack(user_emb.T)
    s_i = _pack(item_emb.T)
    return _sc_call(users2, items2, s_u, s_i)
